# baseline (speedup 1.0000x reference)
import functools

import jax
import jax.numpy as jnp
from jax.experimental import pallas as pl
from jax.experimental.pallas import tpu as pltpu

_TV = 2048
_NSTREAM = 4


def _topk_kern(x_ref, *rest, v_total, tv, nstream):
    w_refs = rest[:nstream]
    b_ref = rest[nstream]
    out_ref = rest[nstream + 1]
    best_val = rest[nstream + 2]
    best_idx = rest[nstream + 3]

    j = pl.program_id(0)
    nj = pl.num_programs(0)

    @pl.when(j == 0)
    def _init():
        best_val[0] = -jnp.inf
        best_idx[0] = 0

    dsub = w_refs[0].shape[0]
    acc = b_ref[...]
    for k in range(nstream):
        xk = x_ref[pl.ds(k * dsub, dsub), :]
        acc = acc + jnp.sum(xk * w_refs[k][...], axis=0, keepdims=True)
    logits = acc
    col = j * tv + jax.lax.broadcasted_iota(jnp.int32, logits.shape, 1)
    logits = jnp.where(col < v_total, logits, -jnp.inf)
    m = jnp.max(logits)
    li = jnp.min(jnp.where(logits == m, col, jnp.iinfo(jnp.int32).max))

    @pl.when(m > best_val[0])
    def _update():
        best_val[0] = m
        best_idx[0] = li

    @pl.when(j == nj - 1)
    def _emit():
        out_ref[0] = best_idx[0]


def kernel(x, W, b):
    d, v = W.shape
    tv = min(_TV, v)
    nj = pl.cdiv(v, tv)
    ns = _NSTREAM if d % _NSTREAM == 0 else 1
    dsub = d // ns
    x0 = x[0:1].reshape(d, 1)
    b2 = b.reshape(1, v)
    w_specs = [
        pl.BlockSpec((dsub, tv), functools.partial(lambda j, k: (k, j), k=k))
        for k in range(ns)
    ]
    out = pl.pallas_call(
        functools.partial(_topk_kern, v_total=v, tv=tv, nstream=ns),
        grid=(nj,),
        in_specs=[pl.BlockSpec((d, 1), lambda j: (0, 0))]
        + w_specs
        + [pl.BlockSpec((1, tv), lambda j: (0, j))],
        out_specs=pl.BlockSpec(memory_space=pltpu.SMEM),
        out_shape=jax.ShapeDtypeStruct((1,), jnp.int32),
        scratch_shapes=[
            pltpu.SMEM((1,), jnp.float32),
            pltpu.SMEM((1,), jnp.int32),
        ],
        compiler_params=pltpu.CompilerParams(
            dimension_semantics=("arbitrary",),
        ),
    )(x0, *([W] * ns), b2)
    return out
